# baseline (speedup 1.0000x reference)
import jax
import jax.numpy as jnp
from jax.experimental import pallas as pl

_N_Y = 8000
_N_X = 2000
_N = _N_Y + _N_X
_NFEAT = 128
_NHID_LOW = 256
_NHID_HIGH = 64
_NCLASS = 40

_BM = 400


def _prep_body(xe_ref, y_ref, fc1wt_ref, fc1b_ref, gc1w_ref, s1_ref):
    gc1w = gc1w_ref[...]
    s1_ref[:_N_Y, :] = jnp.dot(y_ref[...], gc1w, preferred_element_type=jnp.float32)
    x_new = (
        jnp.dot(xe_ref[...], fc1wt_ref[...], preferred_element_type=jnp.float32)
        + fc1b_ref[...]
    )
    s1_ref[_N_Y:, :] = jnp.dot(x_new, gc1w, preferred_element_type=jnp.float32)


def _gc1_body(f_ref, s1_ref, gc1b_ref, gc2w_ref, yemb_ref, s2_ref):
    acc = jnp.dot(f_ref[...], s1_ref[...], preferred_element_type=jnp.float32)
    yemb = jnp.maximum(acc + gc1b_ref[...], 0.0)
    yemb_ref[...] = yemb
    s2_ref[...] = jnp.dot(yemb, gc2w_ref[...], preferred_element_type=jnp.float32)


def _gc2_body(c_ref, s2_ref, gc2b_ref, out_ref):
    logits = (
        jnp.dot(c_ref[...], s2_ref[...], preferred_element_type=jnp.float32)
        + gc2b_ref[...]
    )
    m = jnp.max(logits, axis=1, keepdims=True)
    lse = jnp.log(jnp.sum(jnp.exp(logits - m), axis=1, keepdims=True)) + m
    out_ref[...] = logits - lse


def kernel(X_embedding, Y, F_tilde, C_tilde, fc1_W, fc1_b, gc1_W, gc1_b, gc2_W, gc2_b):
    fc1_Wt = fc1_W.T
    fc1_b2 = fc1_b.reshape(1, _NFEAT)
    gc1_b2 = gc1_b.reshape(1, _NHID_HIGH)
    gc2_b2 = gc2_b.reshape(1, _NCLASS)

    s1 = pl.pallas_call(
        _prep_body,
        out_shape=jax.ShapeDtypeStruct((_N, _NHID_HIGH), jnp.float32),
    )(X_embedding, Y, fc1_Wt, fc1_b2, gc1_W)

    n_blocks = _N // _BM
    yemb, s2 = pl.pallas_call(
        _gc1_body,
        grid=(n_blocks,),
        in_specs=[
            pl.BlockSpec((_BM, _N), lambda i: (i, 0)),
            pl.BlockSpec((_N, _NHID_HIGH), lambda i: (0, 0)),
            pl.BlockSpec((1, _NHID_HIGH), lambda i: (0, 0)),
            pl.BlockSpec((_NHID_HIGH, _NCLASS), lambda i: (0, 0)),
        ],
        out_specs=[
            pl.BlockSpec((_BM, _NHID_HIGH), lambda i: (i, 0)),
            pl.BlockSpec((_BM, _NCLASS), lambda i: (i, 0)),
        ],
        out_shape=[
            jax.ShapeDtypeStruct((_N, _NHID_HIGH), jnp.float32),
            jax.ShapeDtypeStruct((_N, _NCLASS), jnp.float32),
        ],
    )(F_tilde, s1, gc1_b2, gc2_W)

    out = pl.pallas_call(
        _gc2_body,
        grid=(n_blocks,),
        in_specs=[
            pl.BlockSpec((_BM, _N), lambda i: (i, 0)),
            pl.BlockSpec((_N, _NCLASS), lambda i: (0, 0)),
            pl.BlockSpec((1, _NCLASS), lambda i: (0, 0)),
        ],
        out_specs=pl.BlockSpec((_BM, _NCLASS), lambda i: (i, 0)),
        out_shape=jax.ShapeDtypeStruct((_N, _NCLASS), jnp.float32),
    )(C_tilde, s2, gc2_b2)

    return (out, yemb)


# speedup vs baseline: 1.0003x; 1.0003x over previous
import jax
import jax.numpy as jnp
from jax.experimental import pallas as pl

_N_Y = 8000
_N_X = 2000
_N = _N_Y + _N_X
_NFEAT = 128
_NHID_LOW = 256
_NHID_HIGH = 64
_NCLASS = 40

_BM = 400


def _prep_body(xe_ref, y_ref, fc1wt_ref, fc1b_ref, gc1w_ref, s1_ref):
    gc1w = gc1w_ref[...]
    s1_ref[:_N_Y, :] = jnp.dot(y_ref[...], gc1w, preferred_element_type=jnp.float32)
    x_new = (
        jnp.dot(xe_ref[...], fc1wt_ref[...], preferred_element_type=jnp.float32)
        + fc1b_ref[...]
    )
    s1_ref[_N_Y:, :] = jnp.dot(x_new, gc1w, preferred_element_type=jnp.float32)


def _gc1_body(f_ref, s1_ref, gc1b_ref, gc2w_ref, yemb_ref, s2_ref):
    acc = jnp.dot(
        f_ref[...].astype(jnp.bfloat16),
        s1_ref[...].astype(jnp.bfloat16),
        preferred_element_type=jnp.float32,
    )
    yemb = jnp.maximum(acc + gc1b_ref[...], 0.0)
    yemb_ref[...] = yemb
    s2_ref[...] = jnp.dot(yemb, gc2w_ref[...], preferred_element_type=jnp.float32)


def _gc2_body(c_ref, s2_ref, gc2b_ref, out_ref):
    logits = (
        jnp.dot(
            c_ref[...].astype(jnp.bfloat16),
            s2_ref[...].astype(jnp.bfloat16),
            preferred_element_type=jnp.float32,
        )
        + gc2b_ref[...]
    )
    m = jnp.max(logits, axis=1, keepdims=True)
    lse = jnp.log(jnp.sum(jnp.exp(logits - m), axis=1, keepdims=True)) + m
    out_ref[...] = logits - lse


def kernel(X_embedding, Y, F_tilde, C_tilde, fc1_W, fc1_b, gc1_W, gc1_b, gc2_W, gc2_b):
    fc1_Wt = fc1_W.T
    fc1_b2 = fc1_b.reshape(1, _NFEAT)
    gc1_b2 = gc1_b.reshape(1, _NHID_HIGH)
    gc2_b2 = gc2_b.reshape(1, _NCLASS)

    s1 = pl.pallas_call(
        _prep_body,
        out_shape=jax.ShapeDtypeStruct((_N, _NHID_HIGH), jnp.float32),
    )(X_embedding, Y, fc1_Wt, fc1_b2, gc1_W)

    n_blocks = _N // _BM
    yemb, s2 = pl.pallas_call(
        _gc1_body,
        grid=(n_blocks,),
        in_specs=[
            pl.BlockSpec((_BM, _N), lambda i: (i, 0)),
            pl.BlockSpec((_N, _NHID_HIGH), lambda i: (0, 0)),
            pl.BlockSpec((1, _NHID_HIGH), lambda i: (0, 0)),
            pl.BlockSpec((_NHID_HIGH, _NCLASS), lambda i: (0, 0)),
        ],
        out_specs=[
            pl.BlockSpec((_BM, _NHID_HIGH), lambda i: (i, 0)),
            pl.BlockSpec((_BM, _NCLASS), lambda i: (i, 0)),
        ],
        out_shape=[
            jax.ShapeDtypeStruct((_N, _NHID_HIGH), jnp.float32),
            jax.ShapeDtypeStruct((_N, _NCLASS), jnp.float32),
        ],
    )(F_tilde, s1, gc1_b2, gc2_W)

    out = pl.pallas_call(
        _gc2_body,
        grid=(n_blocks,),
        in_specs=[
            pl.BlockSpec((_BM, _N), lambda i: (i, 0)),
            pl.BlockSpec((_N, _NCLASS), lambda i: (0, 0)),
            pl.BlockSpec((1, _NCLASS), lambda i: (0, 0)),
        ],
        out_specs=pl.BlockSpec((_BM, _NCLASS), lambda i: (i, 0)),
        out_shape=jax.ShapeDtypeStruct((_N, _NCLASS), jnp.float32),
    )(C_tilde, s2, gc2_b2)

    return (out, yemb)
